# baseline (speedup 1.0000x reference)
import functools

import jax
import jax.numpy as jnp
from jax import lax
from jax.experimental import pallas as pl
from jax.experimental.pallas import tpu as pltpu
from jax.experimental.pallas import tpu_sc as plsc

N = 4096
K = 32
CUTOFF2 = 2.5 * 2.5
ROWS = 256
WIN = 1536
CAP = 128
NWORK = 32
RPW = N // NWORK
SLAB = 2
INF = float("inf")


def _dist_kernel(w_ref, pos_row_ref, bat_row_ref, pos_t_ref, bat_t_ref,
                 key_ref, d2e_ref, cnt_ref):
    r = pl.program_id(0)
    w0 = pl.multiple_of(w_ref[r], 128)

    xi = pos_row_ref[:, 0:1]
    yi = pos_row_ref[:, 1:2]
    zi = pos_row_ref[:, 2:3]
    pt = pos_t_ref[:, pl.ds(w0, WIN)]
    xj = pt[0:1, :]
    yj = pt[1:2, :]
    zj = pt[2:3, :]

    sqi = xi * xi + yi * yi + zi * zi
    sqj = xj * xj + yj * yj + zj * zj
    dot = jnp.dot(pos_row_ref[:, :], pt,
                  preferred_element_type=jnp.float32)
    d2 = (sqi + sqj) - 2.0 * dot
    d2 = jnp.maximum(d2, 0.0)

    dx = xi - xj
    dy = yi - yj
    dz = zi - zj
    d2e_ref[:, :] = dx * dx + dy * dy + dz * dz

    iota_j = w0 + lax.broadcasted_iota(jnp.int32, (ROWS, WIN), 1)
    row_ids = r * ROWS + lax.broadcasted_iota(jnp.int32, (ROWS, 1), 0)
    same = bat_row_ref[:, 0:1] == bat_t_ref[0:1, pl.ds(w0, WIN)]
    valid = same & (iota_j != row_ids) & (d2 <= CUTOFF2)

    key_ref[:, :] = jnp.where(valid, d2, INF)
    cnt_ref[:, :] = jnp.sum(valid.astype(jnp.int32), axis=1, keepdims=True)


def _dist_call(wstarts, pos, bat_row, pos_t, bat_t):
    grid_spec = pltpu.PrefetchScalarGridSpec(
        num_scalar_prefetch=1,
        grid=(N // ROWS,),
        in_specs=[
            pl.BlockSpec((ROWS, 3), lambda r, w: (r, 0)),
            pl.BlockSpec((ROWS, 1), lambda r, w: (r, 0)),
            pl.BlockSpec((3, N), lambda r, w: (0, 0)),
            pl.BlockSpec((1, N), lambda r, w: (0, 0)),
        ],
        out_specs=[
            pl.BlockSpec((ROWS, WIN), lambda r, w: (r, 0)),
            pl.BlockSpec((ROWS, WIN), lambda r, w: (r, 0)),
            pl.BlockSpec((ROWS, 1), lambda r, w: (r, 0)),
        ],
    )
    return pl.pallas_call(
        _dist_kernel,
        grid_spec=grid_spec,
        out_shape=[
            jax.ShapeDtypeStruct((N, WIN), jnp.float32),
            jax.ShapeDtypeStruct((N, WIN), jnp.float32),
            jax.ShapeDtypeStruct((N, 1), jnp.int32),
        ],
        compiler_params=pltpu.CompilerParams(
            dimension_semantics=("parallel",)),
    )(wstarts, pos, bat_row, pos_t, bat_t)


def _compact_sc(key_hbm, d2e_hbm, ck_hbm, cd_hbm, cc_hbm,
                kslab, dslab, cks, cds, ccs):
    wid = lax.axis_index("s") * 2 + lax.axis_index("c")
    base = wid * RPW
    iota16 = lax.iota(jnp.int32, 16)
    inf16 = jnp.full((16,), jnp.inf, jnp.float32)
    big16 = jnp.full((16,), N, jnp.int32)

    def slab_body(slab, _):
        r0 = base + slab * SLAB
        pltpu.sync_copy(key_hbm.at[pl.ds(r0 * WIN, SLAB * WIN)], kslab)
        pltpu.sync_copy(d2e_hbm.at[pl.ds(r0 * WIN, SLAB * WIN)], dslab)
        for rr in range(SLAB):
            for b in range(CAP // 16):
                cks[pl.ds(rr * CAP + b * 16, 16)] = inf16
                ccs[pl.ds(rr * CAP + b * 16, 16)] = big16
            off = jnp.int32(rr * CAP)
            for c in range(WIN // 16):
                k16 = kslab[pl.ds(rr * WIN + c * 16, 16)]
                msk = k16 < INF
                cnt = jnp.sum(msk.astype(jnp.int32))
                @pl.when(cnt > 0)
                def _(off=off, c=c, rr=rr, k16=k16, msk=msk):
                    pos = off + plsc.cumsum(msk.astype(jnp.int32)) - 1
                    msk2 = msk & (pos < (rr * CAP + CAP))
                    d16 = dslab[pl.ds(rr * WIN + c * 16, 16)]
                    col16 = iota16 + (c * 16)
                    plsc.store_scatter(cks, [pos], k16, mask=msk2)
                    plsc.store_scatter(cds, [pos], d16, mask=msk2)
                    plsc.store_scatter(ccs, [pos], col16, mask=msk2)
                off = off + cnt
        pltpu.sync_copy(cks, ck_hbm.at[pl.ds(r0 * CAP, SLAB * CAP)])
        pltpu.sync_copy(cds, cd_hbm.at[pl.ds(r0 * CAP, SLAB * CAP)])
        pltpu.sync_copy(ccs, cc_hbm.at[pl.ds(r0 * CAP, SLAB * CAP)])
        return 0

    lax.fori_loop(0, RPW // SLAB, slab_body, 0)


def _compact_call(keymat, d2emat):
    mesh = plsc.VectorSubcoreMesh(core_axis_name="c", subcore_axis_name="s")
    fn = functools.partial(
        pl.kernel, mesh=mesh,
        out_type=[
            jax.ShapeDtypeStruct((N * CAP,), jnp.float32),
            jax.ShapeDtypeStruct((N * CAP,), jnp.float32),
            jax.ShapeDtypeStruct((N * CAP,), jnp.int32),
        ],
        compiler_params=pltpu.CompilerParams(needs_layout_passes=False),
        scratch_types=[
            pltpu.VMEM((SLAB * WIN,), jnp.float32),
            pltpu.VMEM((SLAB * WIN,), jnp.float32),
            pltpu.VMEM((SLAB * CAP,), jnp.float32),
            pltpu.VMEM((SLAB * CAP,), jnp.float32),
            pltpu.VMEM((SLAB * CAP,), jnp.int32),
        ],
    )(_compact_sc)
    ck, cd, cc = fn(keymat.reshape(-1), d2emat.reshape(-1))
    return (ck.reshape(N, CAP), cd.reshape(N, CAP), cc.reshape(N, CAP))


def _select_kernel(w_ref, ck_ref, cd_ref, cc_ref, idx_ref, wout_ref):
    r = pl.program_id(0)
    w0 = w_ref[r]

    key = ck_ref[:, :]
    d2e = cd_ref[:, :]
    colf = cc_ref[:, :].astype(jnp.float32)
    row_ids = r * ROWS + lax.broadcasted_iota(jnp.int32, (ROWS, 1), 0)
    row_ids_f = row_ids.astype(jnp.float32)
    w0f = w0.astype(jnp.float32)
    inf = jnp.float32(jnp.inf)
    big = jnp.float32(2 * N)

    for k in range(K):
        m = jnp.min(key, axis=1, keepdims=True)
        hit = key == m
        colm = jnp.min(jnp.where(hit, colf, big), axis=1, keepdims=True)
        sel = colf == colm
        w2 = jnp.min(jnp.where(sel, d2e, inf), axis=1, keepdims=True)
        finite = m < inf
        idx_ref[:, k:k + 1] = jnp.where(
            finite, w0f + colm, row_ids_f).astype(jnp.int32)
        wout_ref[:, k:k + 1] = jnp.where(finite, jnp.sqrt(w2), 0.0)
        key = jnp.where(sel, inf, key)


def _select_call(wstarts, ckey, cd2e, ccol):
    grid_spec = pltpu.PrefetchScalarGridSpec(
        num_scalar_prefetch=1,
        grid=(N // ROWS,),
        in_specs=[
            pl.BlockSpec((ROWS, CAP), lambda r, w: (r, 0)),
            pl.BlockSpec((ROWS, CAP), lambda r, w: (r, 0)),
            pl.BlockSpec((ROWS, CAP), lambda r, w: (r, 0)),
        ],
        out_specs=[
            pl.BlockSpec((ROWS, K), lambda r, w: (r, 0)),
            pl.BlockSpec((ROWS, K), lambda r, w: (r, 0)),
        ],
    )
    return pl.pallas_call(
        _select_kernel,
        grid_spec=grid_spec,
        out_shape=[
            jax.ShapeDtypeStruct((N, K), jnp.int32),
            jax.ShapeDtypeStruct((N, K), jnp.float32),
        ],
        compiler_params=pltpu.CompilerParams(
            dimension_semantics=("parallel",)),
    )(wstarts, ckey, cd2e, ccol)


def _mono_kernel(w_ref, pos_row_ref, bat_row_ref, pos_t_ref, bat_t_ref,
                 idx_ref, wout_ref, *, width):
    r = pl.program_id(0)
    w0 = pl.multiple_of(w_ref[r], 128)

    xi = pos_row_ref[:, 0:1]
    yi = pos_row_ref[:, 1:2]
    zi = pos_row_ref[:, 2:3]
    pt = pos_t_ref[:, pl.ds(w0, width)]
    xj = pt[0:1, :]
    yj = pt[1:2, :]
    zj = pt[2:3, :]

    sqi = xi * xi + yi * yi + zi * zi
    sqj = xj * xj + yj * yj + zj * zj
    dot = jnp.dot(pos_row_ref[:, :], pt, preferred_element_type=jnp.float32)
    d2 = (sqi + sqj) - 2.0 * dot
    d2 = jnp.maximum(d2, 0.0)

    dx = xi - xj
    dy = yi - yj
    dz = zi - zj
    d2e = dx * dx + dy * dy + dz * dz

    iota_j = w0 + lax.broadcasted_iota(jnp.int32, (ROWS, width), 1)
    row_ids = r * ROWS + lax.broadcasted_iota(jnp.int32, (ROWS, 1), 0)
    same = bat_row_ref[:, 0:1] == bat_t_ref[0:1, pl.ds(w0, width)]
    valid = same & (iota_j != row_ids) & (d2 <= CUTOFF2)

    inf = jnp.float32(jnp.inf)
    key = jnp.where(valid, d2, inf)
    iota_f = iota_j.astype(jnp.float32)
    big = jnp.float32(N)
    row_ids_f = row_ids.astype(jnp.float32)
    for k in range(K):
        m = jnp.min(key, axis=1, keepdims=True)
        hit = key == m
        idxm = jnp.min(jnp.where(hit, iota_f, big), axis=1, keepdims=True)
        sel = iota_f == idxm
        w2 = jnp.min(jnp.where(sel, d2e, inf), axis=1, keepdims=True)
        finite = m < inf
        idx_ref[:, k:k + 1] = jnp.where(finite, idxm, row_ids_f).astype(jnp.int32)
        wout_ref[:, k:k + 1] = jnp.where(finite, jnp.sqrt(w2), 0.0)
        key = jnp.where(sel, inf, key)


def _mono_call(width, wstarts, pos, bat_row, pos_t, bat_t):
    grid_spec = pltpu.PrefetchScalarGridSpec(
        num_scalar_prefetch=1,
        grid=(N // ROWS,),
        in_specs=[
            pl.BlockSpec((ROWS, 3), lambda r, w: (r, 0)),
            pl.BlockSpec((ROWS, 1), lambda r, w: (r, 0)),
            pl.BlockSpec((3, N), lambda r, w: (0, 0)),
            pl.BlockSpec((1, N), lambda r, w: (0, 0)),
        ],
        out_specs=[
            pl.BlockSpec((ROWS, K), lambda r, w: (r, 0)),
            pl.BlockSpec((ROWS, K), lambda r, w: (r, 0)),
        ],
    )
    return pl.pallas_call(
        functools.partial(_mono_kernel, width=width),
        grid_spec=grid_spec,
        out_shape=[
            jax.ShapeDtypeStruct((N, K), jnp.int32),
            jax.ShapeDtypeStruct((N, K), jnp.float32),
        ],
        compiler_params=pltpu.CompilerParams(
            dimension_semantics=("parallel",)),
    )(wstarts, pos, bat_row, pos_t, bat_t)


@jax.jit
def kernel(pos, batch):
    bat32 = batch.astype(jnp.int32)
    pos_t = pos.T
    bat_row = bat32.reshape(N, 1)
    bat_t = bat32.reshape(1, N)

    t0 = jnp.arange(N // ROWS, dtype=jnp.int32) * ROWS
    g_lo = bat32[t0]
    g_hi = bat32[t0 + (ROWS - 1)]
    col_lo = jnp.searchsorted(bat32, g_lo, side="left").astype(jnp.int32)
    col_hi = jnp.searchsorted(bat32, g_hi, side="right").astype(jnp.int32)
    wstarts = jnp.minimum((col_lo // 128) * 128, N - WIN)
    fits = jnp.max(col_hi - wstarts) <= WIN
    zeros = jnp.zeros_like(wstarts)

    keymat, d2emat, counts = _dist_call(wstarts, pos, bat_row, pos_t, bat_t)

    def fast():
        ckey, cd2e, ccol = _compact_call(keymat, d2emat)
        return _select_call(wstarts, ckey, cd2e, ccol)

    def slow():
        return _mono_call(N, zeros, pos, bat_row, pos_t, bat_t)

    pred = fits & (jnp.max(counts) <= CAP)
    idx, w = lax.cond(pred, fast, slow)

    tgt = jnp.broadcast_to(jnp.arange(N, dtype=jnp.int32)[:, None], (N, K))
    edge_index = jnp.stack([idx.reshape(-1), tgt.reshape(-1)]).astype(jnp.int64)
    edge_weight = w.reshape(-1)
    return edge_index, edge_weight


# speedup vs baseline: 2.7013x; 2.7013x over previous
import functools

import jax
import jax.numpy as jnp
from jax import lax
from jax.experimental import pallas as pl
from jax.experimental.pallas import tpu as pltpu

N = 4096
K = 32
CUTOFF2 = 2.5 * 2.5
ROWS = 256
WIN = 1536


def _topk_kernel(w_ref, pos_row_ref, bat_row_ref, pos_t_ref, bat_t_ref,
                 idx_ref, wout_ref, *, width):
    r = pl.program_id(0)
    w0 = pl.multiple_of(w_ref[r], 128)

    xi = pos_row_ref[:, 0:1]
    yi = pos_row_ref[:, 1:2]
    zi = pos_row_ref[:, 2:3]
    pt = pos_t_ref[:, pl.ds(w0, width)]
    xj = pt[0:1, :]
    yj = pt[1:2, :]
    zj = pt[2:3, :]

    sqi = xi * xi + yi * yi + zi * zi
    sqj = xj * xj + yj * yj + zj * zj
    dot = jnp.dot(pos_row_ref[:, :], pt,
                  preferred_element_type=jnp.float32)
    d2 = (sqi + sqj) - 2.0 * dot
    d2 = jnp.maximum(d2, 0.0)

    dx = xi - xj
    dy = yi - yj
    dz = zi - zj
    d2e = dx * dx + dy * dy + dz * dz

    iota_j = w0 + lax.broadcasted_iota(jnp.int32, (ROWS, width), 1)
    row_ids = r * ROWS + lax.broadcasted_iota(jnp.int32, (ROWS, 1), 0)
    same = bat_row_ref[:, 0:1] == bat_t_ref[0:1, pl.ds(w0, width)]
    valid = same & (iota_j != row_ids) & (d2 <= CUTOFF2)

    inf = jnp.float32(jnp.inf)
    key = jnp.where(valid, d2, inf)
    iota_f = iota_j.astype(jnp.float32)
    big = jnp.float32(N)

    row_ids_f = row_ids.astype(jnp.float32)
    for k in range(K):
        m = jnp.min(key, axis=1, keepdims=True)
        hit = key == m
        idxm = jnp.min(jnp.where(hit, iota_f, big), axis=1, keepdims=True)
        sel = iota_f == idxm
        w2 = jnp.min(jnp.where(sel, d2e, inf), axis=1, keepdims=True)
        finite = m < inf
        idx_ref[:, k:k + 1] = jnp.where(finite, idxm, row_ids_f).astype(jnp.int32)
        wout_ref[:, k:k + 1] = jnp.where(finite, jnp.sqrt(w2), 0.0)
        key = jnp.where(sel, inf, key)


def _call(width, wstarts, pos, bat_row, pos_t, bat_t):
    grid_spec = pltpu.PrefetchScalarGridSpec(
        num_scalar_prefetch=1,
        grid=(N // ROWS,),
        in_specs=[
            pl.BlockSpec((ROWS, 3), lambda r, w: (r, 0)),
            pl.BlockSpec((ROWS, 1), lambda r, w: (r, 0)),
            pl.BlockSpec((3, N), lambda r, w: (0, 0)),
            pl.BlockSpec((1, N), lambda r, w: (0, 0)),
        ],
        out_specs=[
            pl.BlockSpec((ROWS, K), lambda r, w: (r, 0)),
            pl.BlockSpec((ROWS, K), lambda r, w: (r, 0)),
        ],
    )
    return pl.pallas_call(
        functools.partial(_topk_kernel, width=width),
        grid_spec=grid_spec,
        out_shape=[
            jax.ShapeDtypeStruct((N, K), jnp.int32),
            jax.ShapeDtypeStruct((N, K), jnp.float32),
        ],
        compiler_params=pltpu.CompilerParams(
            dimension_semantics=("parallel",)),
    )(wstarts, pos, bat_row, pos_t, bat_t)


@jax.jit
def kernel(pos, batch):
    bat32 = batch.astype(jnp.int32)
    pos_t = pos.T
    bat_row = bat32.reshape(N, 1)
    bat_t = bat32.reshape(1, N)

    t0 = jnp.arange(N // ROWS, dtype=jnp.int32) * ROWS
    g_lo = bat32[t0]
    g_hi = bat32[t0 + (ROWS - 1)]
    col_lo = jnp.searchsorted(bat32, g_lo, side="left").astype(jnp.int32)
    col_hi = jnp.searchsorted(bat32, g_hi, side="right").astype(jnp.int32)
    wstarts = jnp.minimum((col_lo // 128) * 128, N - WIN)
    fits = jnp.max(col_hi - wstarts) <= WIN
    zeros = jnp.zeros_like(wstarts)

    idx, w = lax.cond(
        fits,
        lambda: _call(WIN, wstarts, pos, bat_row, pos_t, bat_t),
        lambda: _call(N, zeros, pos, bat_row, pos_t, bat_t),
    )

    tgt = jnp.broadcast_to(jnp.arange(N, dtype=jnp.int32)[:, None], (N, K))
    edge_index = jnp.stack([idx.reshape(-1), tgt.reshape(-1)]).astype(jnp.int64)
    edge_weight = w.reshape(-1)
    return edge_index, edge_weight
